# Initial kernel scaffold; baseline (speedup 1.0000x reference)
import jax
import jax.numpy as jnp
from jax.experimental import pallas as pl


def kernel(x, conv_w, conv_b, cls_w, cls_b, reg_w, reg_b):
    raise NotImplementedError("write your pallas kernel here")


# speedup vs baseline: 1.5208x; 1.5208x over previous
import jax
import jax.numpy as jnp
from jax.experimental import pallas as pl
from jax.experimental.pallas import tpu as pltpu

H = 224
W = 224
WP = 256
CIN = 192
CMID = 256
COUT = 54
R = 16
T = H // R


def _rpn_body(a_ref, b_ref, w_ref, wcr_ref, cb_ref, crb_ref, out_ref):
    seg = jnp.concatenate([a_ref[...], b_ref[:, : 2 * WP + 128]], axis=1)
    acc = jnp.zeros((CMID, R * WP), dtype=jnp.float32)
    for dy in range(3):
        for dx in range(3):
            s = dy * WP + dx
            tap = seg[:, s : s + R * WP]
            acc += jnp.dot(w_ref[dy * 3 + dx], tap,
                           preferred_element_type=jnp.float32)
    y = jnp.maximum(acc + cb_ref[...], 0.0).astype(jnp.bfloat16)
    out = jnp.dot(wcr_ref[...], y, preferred_element_type=jnp.float32)
    out_ref[...] = out + crb_ref[...]


def kernel(x, conv_w, conv_b, cls_w, cls_b, reg_w, reg_b):
    x3 = x.reshape(CIN, H, W)
    xp = jnp.pad(x3, ((0, 0), (1, (T + 1) * R - H - 1), (1, WP - W - 1)))
    xf = xp.astype(jnp.bfloat16).reshape(CIN, (T + 1) * R * WP)

    wt = conv_w.transpose(2, 3, 0, 1).reshape(9, CMID, CIN).astype(jnp.bfloat16)
    wcr = jnp.concatenate(
        [cls_w.reshape(-1, CMID), reg_w.reshape(-1, CMID)]).astype(jnp.bfloat16)
    cb = conv_b.reshape(CMID, 1)
    crb = jnp.concatenate([cls_b, reg_b]).reshape(COUT, 1)

    out = pl.pallas_call(
        _rpn_body,
        grid=(T,),
        in_specs=[
            pl.BlockSpec((CIN, R * WP), lambda i: (0, i)),
            pl.BlockSpec((CIN, R * WP), lambda i: (0, i + 1)),
            pl.BlockSpec((9, CMID, CIN), lambda i: (0, 0, 0)),
            pl.BlockSpec((COUT, CMID), lambda i: (0, 0)),
            pl.BlockSpec((CMID, 1), lambda i: (0, 0)),
            pl.BlockSpec((COUT, 1), lambda i: (0, 0)),
        ],
        out_specs=pl.BlockSpec((COUT, R * WP), lambda i: (0, i)),
        out_shape=jax.ShapeDtypeStruct((COUT, H * WP), jnp.float32),
        compiler_params=pltpu.CompilerParams(
            dimension_semantics=("arbitrary",)),
    )(xf, xf, wt, wcr, cb, crb)

    out = out.reshape(COUT, H, WP)[:, :, :W]
    cls_out = out[:18].reshape(1, 18, H, W)
    reg_out = out[18:].reshape(1, 36, H, W)
    return (cls_out, reg_out)


# speedup vs baseline: 1.9984x; 1.3140x over previous
import jax
import jax.numpy as jnp
from jax.experimental import pallas as pl
from jax.experimental.pallas import tpu as pltpu

H = 224
W = 224
WP = 256
CIN = 192
CMID = 256
R = 16
T = H // R
M = R * WP
SEG = (R + 2) * WP + 128


def _rpn_body(a_ref, b_ref, w_ref, wcr_ref, cb_ref, crb_ref,
              cls_ref, reg_ref, seg_ref, top_ref):
    i = pl.program_id(0)

    cen = a_ref[...].astype(jnp.bfloat16)
    nxt = b_ref[:, :W].astype(jnp.bfloat16)
    top = jnp.where(i == 0, jnp.zeros_like(top_ref), top_ref[...])
    nxt = jnp.where(i == T - 1, jnp.zeros_like(nxt), nxt)

    @pl.when(i == 0)
    def _():
        seg_ref[...] = jnp.zeros((CIN, SEG), jnp.bfloat16)

    seg_ref[:, 1:W + 1] = top
    for j in range(R):
        seg_ref[:, (j + 1) * WP + 1:(j + 1) * WP + 1 + W] = \
            cen[:, j * W:(j + 1) * W]
    seg_ref[:, (R + 1) * WP + 1:(R + 1) * WP + 1 + W] = nxt

    seg = seg_ref[...]
    acc = jnp.zeros((CMID, M), dtype=jnp.float32)
    for dy in range(3):
        for dx in range(3):
            s = dy * WP + dx
            acc += jnp.dot(w_ref[dy * 3 + dx], seg[:, s:s + M],
                           preferred_element_type=jnp.float32)
    y = jnp.maximum(acc + cb_ref[...], 0.0).astype(jnp.bfloat16)
    o = jnp.dot(wcr_ref[...], y, preferred_element_type=jnp.float32)
    o = o + crb_ref[...]
    for r in range(R):
        row = o[:, r * WP:r * WP + W]
        cls_ref[:, r, :] = row[:18]
        reg_ref[:, r, :] = row[18:]

    top_ref[...] = cen[:, (R - 1) * W:]


def kernel(x, conv_w, conv_b, cls_w, cls_b, reg_w, reg_b):
    xf = x.reshape(CIN, H * W)
    wt = conv_w.transpose(2, 3, 0, 1).reshape(9, CMID, CIN).astype(jnp.bfloat16)
    wcr = jnp.concatenate(
        [cls_w.reshape(-1, CMID), reg_w.reshape(-1, CMID)]).astype(jnp.bfloat16)
    cb = conv_b.reshape(CMID, 1)
    crb = jnp.concatenate([cls_b, reg_b]).reshape(54, 1)

    cls_out, reg_out = pl.pallas_call(
        _rpn_body,
        grid=(T,),
        in_specs=[
            pl.BlockSpec((CIN, R * W), lambda i: (0, i)),
            pl.BlockSpec((CIN, R * W), lambda i: (0, jnp.minimum(i + 1, T - 1))),
            pl.BlockSpec((9, CMID, CIN), lambda i: (0, 0, 0)),
            pl.BlockSpec((54, CMID), lambda i: (0, 0)),
            pl.BlockSpec((CMID, 1), lambda i: (0, 0)),
            pl.BlockSpec((54, 1), lambda i: (0, 0)),
        ],
        out_specs=[
            pl.BlockSpec((18, R, W), lambda i: (0, i, 0)),
            pl.BlockSpec((36, R, W), lambda i: (0, i, 0)),
        ],
        out_shape=[
            jax.ShapeDtypeStruct((18, H, W), jnp.float32),
            jax.ShapeDtypeStruct((36, H, W), jnp.float32),
        ],
        scratch_shapes=[
            pltpu.VMEM((CIN, SEG), jnp.bfloat16),
            pltpu.VMEM((CIN, W), jnp.bfloat16),
        ],
        compiler_params=pltpu.CompilerParams(
            dimension_semantics=("arbitrary",)),
    )(xf, xf, wt, wcr, cb, crb)

    return (cls_out.reshape(1, 18, H, W), reg_out.reshape(1, 36, H, W))


# speedup vs baseline: 2.5903x; 1.2962x over previous
import jax
import jax.numpy as jnp
from jax.experimental import pallas as pl
from jax.experimental.pallas import tpu as pltpu

H = 224
W = 224
WP = 256
CIN = 192
CMID = 256
R = 16
T = H // R
M = R * WP
SEG = (R + 2) * WP + 128


def _rpn_body(a_ref, b_ref, w_ref, wcr_ref, cb_ref, crb_ref,
              cls_ref, reg_ref, seg_ref, top_ref):
    i = pl.program_id(0)

    cen = jnp.transpose(a_ref[...].astype(jnp.bfloat16), (1, 0, 2))
    nxt = b_ref[:, 0, :].astype(jnp.bfloat16)
    top = jnp.where(i == 0, jnp.zeros_like(top_ref), top_ref[...])
    nxt = jnp.where(i == T - 1, jnp.zeros_like(nxt), nxt)

    @pl.when(i == 0)
    def _():
        seg_ref[...] = jnp.zeros((CIN, SEG), jnp.bfloat16)

    seg_ref[:, 1:W + 1] = top
    for j in range(R):
        seg_ref[:, (j + 1) * WP + 1:(j + 1) * WP + 1 + W] = cen[j]
    seg_ref[:, (R + 1) * WP + 1:(R + 1) * WP + 1 + W] = nxt

    seg = seg_ref[...]
    acc = jnp.zeros((CMID, M), dtype=jnp.float32)
    for dy in range(3):
        for dx in range(3):
            s = dy * WP + dx
            acc += jnp.dot(w_ref[dy * 3 + dx], seg[:, s:s + M],
                           preferred_element_type=jnp.float32)
    y = jnp.maximum(acc + cb_ref[...], 0.0).astype(jnp.bfloat16)
    o = jnp.dot(wcr_ref[...], y, preferred_element_type=jnp.float32)
    o = o + crb_ref[...]
    for r in range(R):
        row = o[:, r * WP:r * WP + W]
        cls_ref[:, r, :] = row[:18]
        reg_ref[:, r, :] = row[18:]

    top_ref[...] = cen[R - 1]


def kernel(x, conv_w, conv_b, cls_w, cls_b, reg_w, reg_b):
    x3 = x.reshape(CIN, H, W)
    wt = conv_w.transpose(2, 3, 0, 1).reshape(9, CMID, CIN).astype(jnp.bfloat16)
    wcr = jnp.concatenate(
        [cls_w.reshape(-1, CMID), reg_w.reshape(-1, CMID)]).astype(jnp.bfloat16)
    cb = conv_b.reshape(CMID, 1)
    crb = jnp.concatenate([cls_b, reg_b]).reshape(54, 1)

    cls_out, reg_out = pl.pallas_call(
        _rpn_body,
        grid=(T,),
        in_specs=[
            pl.BlockSpec((CIN, R, W), lambda i: (0, i, 0)),
            pl.BlockSpec((CIN, R, W), lambda i: (0, jnp.minimum(i + 1, T - 1), 0)),
            pl.BlockSpec((9, CMID, CIN), lambda i: (0, 0, 0)),
            pl.BlockSpec((54, CMID), lambda i: (0, 0)),
            pl.BlockSpec((CMID, 1), lambda i: (0, 0)),
            pl.BlockSpec((54, 1), lambda i: (0, 0)),
        ],
        out_specs=[
            pl.BlockSpec((18, R, W), lambda i: (0, i, 0)),
            pl.BlockSpec((36, R, W), lambda i: (0, i, 0)),
        ],
        out_shape=[
            jax.ShapeDtypeStruct((18, H, W), jnp.float32),
            jax.ShapeDtypeStruct((36, H, W), jnp.float32),
        ],
        scratch_shapes=[
            pltpu.VMEM((CIN, SEG), jnp.bfloat16),
            pltpu.VMEM((CIN, W), jnp.bfloat16),
        ],
        compiler_params=pltpu.CompilerParams(
            dimension_semantics=("arbitrary",)),
    )(x3, x3, wt, wcr, cb, crb)

    return (cls_out.reshape(1, 18, H, W), reg_out.reshape(1, 36, H, W))


# speedup vs baseline: 3.3080x; 1.2771x over previous
import jax
import jax.numpy as jnp
from jax.experimental import pallas as pl
from jax.experimental.pallas import tpu as pltpu

H = 224
W = 224
WP = 256
CIN = 192
CMID = 256
R = 16
T = H // R
M = R * WP
SEG = (R + 2) * WP + 128


def _rpn_body(a_ref, b_ref, w_ref, wcr_ref, cb_ref, crb_ref,
              cls_ref, reg_ref, seg_ref, top_ref):
    i = pl.program_id(0)

    cen = jnp.transpose(a_ref[...].astype(jnp.bfloat16), (1, 0, 2))
    nxt = b_ref[:, 0, :].astype(jnp.bfloat16)
    top = jnp.where(i == 0, jnp.zeros_like(top_ref), top_ref[...])
    nxt = jnp.where(i == T - 1, jnp.zeros_like(nxt), nxt)

    @pl.when(i == 0)
    def _():
        seg_ref[...] = jnp.zeros((CIN, SEG), jnp.bfloat16)

    seg_ref[:, 1:W + 1] = top
    for j in range(R):
        seg_ref[:, (j + 1) * WP + 1:(j + 1) * WP + 1 + W] = cen[j]
    seg_ref[:, (R + 1) * WP + 1:(R + 1) * WP + 1 + W] = nxt

    seg = seg_ref[...]
    taps = jnp.concatenate(
        [seg[:, dy * WP + dx:dy * WP + dx + M]
         for dy in range(3) for dx in range(3)], axis=0)
    acc = jnp.dot(w_ref[...], taps, preferred_element_type=jnp.float32)
    y = jnp.maximum(acc + cb_ref[...], 0.0).astype(jnp.bfloat16)
    o = jnp.dot(wcr_ref[...], y, preferred_element_type=jnp.float32)
    o = o + crb_ref[...]
    for r in range(R):
        row = o[:, r * WP:r * WP + W]
        cls_ref[:, r, :] = row[:18]
        reg_ref[:, r, :] = row[18:]

    top_ref[...] = cen[R - 1]


def kernel(x, conv_w, conv_b, cls_w, cls_b, reg_w, reg_b):
    x3 = x.reshape(CIN, H, W)
    wt = conv_w.transpose(0, 2, 3, 1).reshape(CMID, 9 * CIN).astype(jnp.bfloat16)
    wcr = jnp.concatenate(
        [cls_w.reshape(-1, CMID), reg_w.reshape(-1, CMID)]).astype(jnp.bfloat16)
    cb = conv_b.reshape(CMID, 1)
    crb = jnp.concatenate([cls_b, reg_b]).reshape(54, 1)

    cls_out, reg_out = pl.pallas_call(
        _rpn_body,
        grid=(T,),
        in_specs=[
            pl.BlockSpec((CIN, R, W), lambda i: (0, i, 0)),
            pl.BlockSpec((CIN, R, W), lambda i: (0, jnp.minimum(i + 1, T - 1), 0)),
            pl.BlockSpec((CMID, 9 * CIN), lambda i: (0, 0)),
            pl.BlockSpec((54, CMID), lambda i: (0, 0)),
            pl.BlockSpec((CMID, 1), lambda i: (0, 0)),
            pl.BlockSpec((54, 1), lambda i: (0, 0)),
        ],
        out_specs=[
            pl.BlockSpec((18, R, W), lambda i: (0, i, 0)),
            pl.BlockSpec((36, R, W), lambda i: (0, i, 0)),
        ],
        out_shape=[
            jax.ShapeDtypeStruct((18, H, W), jnp.float32),
            jax.ShapeDtypeStruct((36, H, W), jnp.float32),
        ],
        scratch_shapes=[
            pltpu.VMEM((CIN, SEG), jnp.bfloat16),
            pltpu.VMEM((CIN, W), jnp.bfloat16),
        ],
        compiler_params=pltpu.CompilerParams(
            dimension_semantics=("arbitrary",)),
    )(x3, x3, wt, wcr, cb, crb)

    return (cls_out.reshape(1, 18, H, W), reg_out.reshape(1, 36, H, W))
